# baseline (speedup 1.0000x reference)
import jax
import jax.numpy as jnp
from jax.experimental import pallas as pl
from jax.experimental.pallas import tpu as pltpu

_MODALITY = (197, 50, 50, 50, 200, 105, 277, 61, 34)
_B, _S, _H = 16, 1024, 1024
_NTYPE = 9
_NTYPE_PAD = 16
_EPS = 1e-12
_POS_ROWS = 280
_GROUP = 2
_ROWS = _GROUP * _S
_NBLK = _B // _GROUP


def _fused_kernel(tt_ref, x_ref, type_ref, pos_ref, gamma_ref, beta_ref,
                  o_ref, pos_emb_ref):
    blk = pl.program_id(0)

    @pl.when(blk == 0)
    def _fill_pos():
        for g in range(_GROUP):
            off = g * _S
            for L in _MODALITY:
                pos_emb_ref[off:off + L, :] = pos_ref[0:L, :]
                off += L

    x = x_ref[0]
    ids = tt_ref[0]
    iota = jax.lax.broadcasted_iota(jnp.int32, (_ROWS, _NTYPE_PAD), 1)
    onehot = (iota == ids).astype(jnp.float32)
    type_emb = jax.lax.dot_general(
        onehot, type_ref[...],
        dimension_numbers=(((1,), (0,)), ((), ())),
        preferred_element_type=jnp.float32)

    s = x + type_emb + pos_emb_ref[...]
    mean = jnp.mean(s, axis=1, keepdims=True)
    var = jnp.mean(s * s, axis=1, keepdims=True) - mean * mean
    inv = jax.lax.rsqrt(var + _EPS)
    o_ref[0] = (s - mean) * inv * gamma_ref[...] + beta_ref[...]


def kernel(inputs_embeds, token_type_ids, pos_table, type_table, ln_gamma, ln_beta):
    x = inputs_embeds.reshape(_NBLK, _ROWS, _H)
    tt = token_type_ids.astype(jnp.int32).reshape(_NBLK, _ROWS, 1)
    type_pad = jnp.zeros((_NTYPE_PAD, _H), jnp.float32).at[:_NTYPE].set(
        type_table.astype(jnp.float32))
    gamma = ln_gamma.reshape(1, _H)
    beta = ln_beta.reshape(1, _H)

    out = pl.pallas_call(
        _fused_kernel,
        grid=(_NBLK,),
        in_specs=[
            pl.BlockSpec((1, _ROWS, 1), lambda b: (b, 0, 0)),
            pl.BlockSpec((1, _ROWS, _H), lambda b: (b, 0, 0)),
            pl.BlockSpec((_NTYPE_PAD, _H), lambda b: (0, 0)),
            pl.BlockSpec((_POS_ROWS, _H), lambda b: (0, 0)),
            pl.BlockSpec((1, _H), lambda b: (0, 0)),
            pl.BlockSpec((1, _H), lambda b: (0, 0)),
        ],
        out_specs=pl.BlockSpec((1, _ROWS, _H), lambda b: (b, 0, 0)),
        out_shape=jax.ShapeDtypeStruct((_NBLK, _ROWS, _H), jnp.float32),
        scratch_shapes=[pltpu.VMEM((_ROWS, _H), jnp.float32)],
    )(tt, x, type_pad, pos_table[:_POS_ROWS], gamma, beta)
    return out.reshape(_B, _S, _H)


# speedup vs baseline: 1.2200x; 1.2200x over previous
import jax
import jax.numpy as jnp
from jax.experimental import pallas as pl
from jax.experimental.pallas import tpu as pltpu

_MODALITY = (197, 50, 50, 50, 200, 105, 277, 61, 34)
_B, _S, _H = 16, 1024, 1024
_NTYPE = 9
_NTYPE_PAD = 16
_EPS = 1e-12
_POS_ROWS = 280
_GROUP = 2
_ROWS = _GROUP * _S
_NBLK = _B // _GROUP


def _fused_kernel(tt_ref, x_ref, type_ref, pos_ref, gamma_ref, beta_ref,
                  o_ref, pos_emb_ref):
    blk = pl.program_id(0)

    @pl.when(blk == 0)
    def _fill_pos():
        for g in range(_GROUP):
            off = g * _S
            for L in _MODALITY:
                pos_emb_ref[off:off + L, :] = pos_ref[0:L, :]
                off += L

    x = x_ref[0]
    ids = tt_ref[0]
    iota = jax.lax.broadcasted_iota(jnp.int32, (_NTYPE_PAD, _ROWS), 0)
    onehot = (iota == ids).astype(jnp.float32)
    type_emb = jax.lax.dot_general(
        onehot, type_ref[...],
        dimension_numbers=(((0,), (0,)), ((), ())),
        preferred_element_type=jnp.float32)

    s = x + type_emb + pos_emb_ref[...]
    mean = jnp.mean(s, axis=1, keepdims=True)
    var = jnp.mean(s * s, axis=1, keepdims=True) - mean * mean
    inv = jax.lax.rsqrt(var + _EPS)
    o_ref[0] = (s - mean) * inv * gamma_ref[...] + beta_ref[...]


def kernel(inputs_embeds, token_type_ids, pos_table, type_table, ln_gamma, ln_beta):
    x = inputs_embeds.reshape(_NBLK, _ROWS, _H)
    tt = token_type_ids.astype(jnp.int32).reshape(_NBLK, 1, _ROWS)
    type_pad = jnp.zeros((_NTYPE_PAD, _H), jnp.float32).at[:_NTYPE].set(
        type_table.astype(jnp.float32))
    gamma = ln_gamma.reshape(1, _H)
    beta = ln_beta.reshape(1, _H)

    out = pl.pallas_call(
        _fused_kernel,
        grid=(_NBLK,),
        in_specs=[
            pl.BlockSpec((1, 1, _ROWS), lambda b: (b, 0, 0)),
            pl.BlockSpec((1, _ROWS, _H), lambda b: (b, 0, 0)),
            pl.BlockSpec((_NTYPE_PAD, _H), lambda b: (0, 0)),
            pl.BlockSpec((_POS_ROWS, _H), lambda b: (0, 0)),
            pl.BlockSpec((1, _H), lambda b: (0, 0)),
            pl.BlockSpec((1, _H), lambda b: (0, 0)),
        ],
        out_specs=pl.BlockSpec((1, _ROWS, _H), lambda b: (b, 0, 0)),
        out_shape=jax.ShapeDtypeStruct((_NBLK, _ROWS, _H), jnp.float32),
        scratch_shapes=[pltpu.VMEM((_ROWS, _H), jnp.float32)],
    )(tt, x, type_pad, pos_table[:_POS_ROWS], gamma, beta)
    return out.reshape(_B, _S, _H)
